# baseline (speedup 1.0000x reference)
import jax
import jax.numpy as jnp
from jax import lax
from jax.experimental import pallas as pl
from jax.experimental.pallas import tpu as pltpu


def _row_tile(h):
    r = min(16, h)
    while h % r:
        r //= 2
    return r


def _act(x, act):
    if act == "relu":
        return jax.nn.relu(x)
    if act == "sigmoid":
        return jax.nn.sigmoid(x)
    return x


def _dimsem(n):
    return pltpu.CompilerParams(
        dimension_semantics=("parallel",) * n,
        vmem_limit_bytes=100 * 1024 * 1024,
    )


def conv3x3_s1(x, w, b, act="relu", w2=None, b2=None, act2=None):
    B_, H_, W_, C_ = x.shape
    O_ = w.shape[0]
    R = _row_tile(H_)
    xp = jnp.pad(x, ((0, 0), (1, 1), (1, 1), (0, 0)))
    rows = [xp[:, d:d + H_] for d in range(3)]
    wk = jnp.transpose(w, (2, 3, 1, 0))
    bb = b.reshape(1, O_)
    fused = w2 is not None
    if fused:
        O2 = w2.shape[0]
        wk2 = w2.reshape(O2, O_).T
        bb2 = b2.reshape(1, O2)
    Oout = O2 if fused else O_

    def kern(x0, x1, x2, wr, br, *rest):
        if fused:
            w2r, b2r, out = rest
        else:
            (out,) = rest
        srcs = (x0, x1, x2)
        acc = jnp.zeros((R * W_, O_), jnp.float32)
        for dy in range(3):
            for dx in range(3):
                a = srcs[dy][0, :, dx:dx + W_, :].reshape(R * W_, C_)
                acc = acc + jnp.dot(a, wr[dy, dx],
                                    preferred_element_type=jnp.float32)
        acc = _act(acc + br[...], act)
        if fused:
            acc = _act(jnp.dot(acc, w2r[...],
                               preferred_element_type=jnp.float32) + b2r[...],
                       act2)
        out[0] = acc.reshape(R, W_, Oout)

    in_specs = [pl.BlockSpec((1, R, W_ + 2, C_), lambda bi, i: (bi, i, 0, 0))
                for _ in range(3)]
    in_specs.append(pl.BlockSpec((3, 3, C_, O_), lambda bi, i: (0, 0, 0, 0)))
    in_specs.append(pl.BlockSpec((1, O_), lambda bi, i: (0, 0)))
    args = rows + [wk, bb]
    if fused:
        in_specs.append(pl.BlockSpec((O_, O2), lambda bi, i: (0, 0)))
        in_specs.append(pl.BlockSpec((1, O2), lambda bi, i: (0, 0)))
        args += [wk2, bb2]
    return pl.pallas_call(
        kern,
        grid=(B_, H_ // R),
        in_specs=in_specs,
        out_specs=pl.BlockSpec((1, R, W_, Oout), lambda bi, i: (bi, i, 0, 0)),
        out_shape=jax.ShapeDtypeStruct((B_, H_, W_, Oout), jnp.float32),
        compiler_params=_dimsem(2),
    )(*args)


def conv3x3_s2(x, w, b):
    B_, H_, W_, C_ = x.shape
    O_ = w.shape[0]
    Ho, Wo = H_ // 2, W_ // 2
    R = _row_tile(Ho)
    xp = jnp.pad(x, ((0, 0), (0, 1), (0, 1), (0, 0)))
    taps = [xp[:, dy:dy + H_:2, dx:dx + W_:2, :]
            for dy in range(3) for dx in range(3)]
    t9 = jnp.concatenate(taps, axis=3)
    wk = jnp.transpose(w, (2, 3, 1, 0)).reshape(9 * C_, O_)
    bb = b.reshape(1, O_)

    def kern(tr, wr, br, out):
        a = tr[0].reshape(R * Wo, 9 * C_)
        acc = jnp.dot(a, wr[...], preferred_element_type=jnp.float32)
        out[0] = jax.nn.relu(acc + br[...]).reshape(R, Wo, O_)

    return pl.pallas_call(
        kern,
        grid=(B_, Ho // R),
        in_specs=[
            pl.BlockSpec((1, R, Wo, 9 * C_), lambda bi, i: (bi, i, 0, 0)),
            pl.BlockSpec((9 * C_, O_), lambda bi, i: (0, 0)),
            pl.BlockSpec((1, O_), lambda bi, i: (0, 0)),
        ],
        out_specs=pl.BlockSpec((1, R, Wo, O_), lambda bi, i: (bi, i, 0, 0)),
        out_shape=jax.ShapeDtypeStruct((B_, Ho, Wo, O_), jnp.float32),
        compiler_params=_dimsem(2),
    )(t9, wk, bb)


def conv3x3_c1_planar(xg, w, b, out_planes=True):
    B_, H_, W_ = xg.shape
    O_ = w.shape[0]
    R = _row_tile(H_)
    xp = jnp.pad(xg, ((0, 0), (1, 1), (1, 1)))
    rows = [xp[:, d:d + H_] for d in range(3)]
    wf = w.reshape(O_, 9)

    def kern(x0, x1, x2, wr, br, out):
        srcs = (x0, x1, x2)
        taps = [srcs[dy][0][:, dx:dx + W_] for dy in range(3) for dx in range(3)]
        for o in range(O_):
            acc = taps[0] * wr[o, 0]
            for k in range(1, 9):
                acc = acc + taps[k] * wr[o, k]
            out[0, o] = jax.nn.relu(acc + br[o])

    return pl.pallas_call(
        kern,
        grid=(B_, H_ // R),
        in_specs=[pl.BlockSpec((1, R, W_ + 2), lambda bi, i: (bi, i, 0))
                  for _ in range(3)]
        + [pl.BlockSpec(memory_space=pltpu.SMEM),
           pl.BlockSpec(memory_space=pltpu.SMEM)],
        out_specs=pl.BlockSpec((1, O_, R, W_), lambda bi, i: (bi, 0, i, 0)),
        out_shape=jax.ShapeDtypeStruct((B_, O_, H_, W_), jnp.float32),
        compiler_params=_dimsem(2),
    )(rows[0], rows[1], rows[2], wf, b)


def attention_refine(mask, w1, b1, w2, b2):
    B_, H_, W_ = mask.shape
    O_ = w1.shape[0]
    R = _row_tile(H_)
    xp = jnp.pad(mask, ((0, 0), (1, 1), (1, 1)))
    rows = [xp[:, d:d + H_] for d in range(3)]
    wf = w1.reshape(O_, 9)
    w2f = w2.reshape(O_)

    def kern(x0, x1, x2, wr, br, w2r, b2r, out):
        srcs = (x0, x1, x2)
        taps = [srcs[dy][0][:, dx:dx + W_] for dy in range(3) for dx in range(3)]
        att = jnp.zeros((R, W_), jnp.float32) + b2r[0]
        for o in range(O_):
            acc = taps[0] * wr[o, 0]
            for k in range(1, 9):
                acc = acc + taps[k] * wr[o, k]
            att = att + jax.nn.relu(acc + br[o]) * w2r[o]
        out[0] = taps[4] * jax.nn.sigmoid(att)

    return pl.pallas_call(
        kern,
        grid=(B_, H_ // R),
        in_specs=[pl.BlockSpec((1, R, W_ + 2), lambda bi, i: (bi, i, 0))
                  for _ in range(3)]
        + [pl.BlockSpec(memory_space=pltpu.SMEM)] * 4,
        out_specs=pl.BlockSpec((1, R, W_), lambda bi, i: (bi, i, 0)),
        out_shape=jax.ShapeDtypeStruct((B_, H_, W_), jnp.float32),
        compiler_params=_dimsem(2),
    )(rows[0], rows[1], rows[2], wf, b1, w2f, b2)


def maxpool2(x):
    B_, H_, W_, C_ = x.shape
    Ho, Wo = H_ // 2, W_ // 2
    R = _row_tile(Ho)
    views = [x[:, dy::2, dx::2, :] for dy in range(2) for dx in range(2)]

    def kern(a, b, c, d, out):
        out[...] = jnp.maximum(jnp.maximum(a[...], b[...]),
                               jnp.maximum(c[...], d[...]))

    return pl.pallas_call(
        kern,
        grid=(B_, Ho // R),
        in_specs=[pl.BlockSpec((1, R, Wo, C_), lambda bi, i: (bi, i, 0, 0))
                  for _ in range(4)],
        out_specs=pl.BlockSpec((1, R, Wo, C_), lambda bi, i: (bi, i, 0, 0)),
        out_shape=jax.ShapeDtypeStruct((B_, Ho, Wo, C_), jnp.float32),
        compiler_params=_dimsem(2),
    )(*views)


def tail(f_gap_in, mask, ph, pe_fw, pe_fb):
    B_, S_, F_ = f_gap_in.shape
    H_, W_ = mask.shape[1], mask.shape[2]
    fwT = pe_fw.T
    fbb = pe_fb.reshape(1, 4)
    w1T = ph['w1'].T
    w2T = ph['w2'].T
    w3T = ph['w3'].T
    b1 = ph['b1'].reshape(1, -1); g1 = ph['g1'].reshape(1, -1)
    be1 = ph['be1'].reshape(1, -1)
    b2 = ph['b2'].reshape(1, -1); g2 = ph['g2'].reshape(1, -1)
    be2 = ph['be2'].reshape(1, -1)
    b3 = ph['b3'].reshape(1, 4)

    def kern(fr, mr, fwr, fbr, w1r, b1r, g1r, be1r, w2r, b2r, g2r, be2r,
             w3r, b3r, out):
        gap = jnp.sum(fr[...], axis=1) * (1.0 / S_)
        bbox_feat = jax.nn.sigmoid(
            jnp.dot(gap, fwr[...], preferred_element_type=jnp.float32)
            + fbr[...])

        m = jnp.where(mr[...] > 0.5, 1.0, 0.0)
        rows = jnp.max(m, axis=2)
        cols = jnp.max(m, axis=1)
        ih = lax.broadcasted_iota(jnp.int32, (B_, H_), 1).astype(jnp.float32)
        iw = lax.broadcasted_iota(jnp.int32, (B_, W_), 1).astype(jnp.float32)
        big = jnp.float32(1e9)
        rmin = jnp.min(jnp.where(rows > 0.5, ih, big), axis=1, keepdims=True)
        rmax = jnp.max(jnp.where(rows > 0.5, ih, -big), axis=1, keepdims=True)
        cmin = jnp.min(jnp.where(cols > 0.5, iw, big), axis=1, keepdims=True)
        cmax = jnp.max(jnp.where(cols > 0.5, iw, -big), axis=1, keepdims=True)
        valid = (jnp.max(rows, axis=1, keepdims=True)
                 * jnp.max(cols, axis=1, keepdims=True)) > 0.5
        bbox = jnp.concatenate(
            [cmin * (1.0 / W_), rmin * (1.0 / H_),
             cmax * (1.0 / W_), rmax * (1.0 / H_)], axis=1)
        bbox = jnp.clip(bbox, 0.0, 1.0)
        mask_bbox = jnp.where(valid, bbox, 0.0)

        def bnorm(h, g, be):
            mu = jnp.mean(h, axis=0, keepdims=True)
            var = jnp.mean((h - mu) * (h - mu), axis=0, keepdims=True)
            return g * (h - mu) * jax.lax.rsqrt(var + 1e-5) + be

        hdd = jnp.concatenate([bbox_feat, mask_bbox], axis=1)
        h1 = bnorm(jax.nn.relu(
            jnp.dot(hdd, w1r[...], preferred_element_type=jnp.float32)
            + b1r[...]), g1r[...], be1r[...])
        h2 = bnorm(jax.nn.relu(
            jnp.dot(h1, w2r[...], preferred_element_type=jnp.float32)
            + b2r[...]), g2r[...], be2r[...])
        fb = jax.nn.sigmoid(
            jnp.dot(h2, w3r[...], preferred_element_type=jnp.float32)
            + b3r[...])
        x1 = fb[:, 0:1]; y1 = fb[:, 1:2]; x2 = fb[:, 2:3]; y2 = fb[:, 3:4]
        x2 = x1 + jax.nn.relu(x2 - x1) + 0.001
        y2 = y1 + jax.nn.relu(y2 - y1) + 0.001
        out[...] = jnp.concatenate([x1, y1, x2, y2], axis=1)

    return pl.pallas_call(
        kern,
        out_shape=jax.ShapeDtypeStruct((B_, 4), jnp.float32),
        compiler_params=pltpu.CompilerParams(
            vmem_limit_bytes=100 * 1024 * 1024),
    )(f_gap_in, mask, fwT, fbb, w1T, b1, g1, be1, w2T, b2, g2, be2, w3T, b3)


def kernel(x, params):
    B_, C_, H_, W_ = x.shape
    pe, pu = params['eff'], params['unet']
    pa, pb, ph = params['att'], params['br'], params['head']

    xt = jnp.transpose(x, (0, 2, 3, 1))
    f = conv3x3_s2(xt, pe['w1'], pe['b1'])
    f = conv3x3_s2(f, pe['w2'], pe['b2'])
    f = conv3x3_s2(f, pe['w3'], pe['b3'])
    S = f.shape[1] * f.shape[2]
    f_gap = f.reshape(B_, S, 128)

    xg = x[:, 0]
    e1p = conv3x3_c1_planar(xg, pu['e1w'], pu['e1b'])
    e1 = jnp.transpose(e1p, (0, 2, 3, 1))
    e2 = conv3x3_s1(maxpool2(e1), pu['e2w'], pu['e2b'])
    bt = conv3x3_s1(maxpool2(e2), pu['bw'], pu['bb'])
    u2 = jnp.repeat(jnp.repeat(bt, 2, axis=1), 2, axis=2)
    d2 = conv3x3_s1(jnp.concatenate([u2, e2], axis=3),
                    pu['d2w'], pu['d2b'])
    u1 = jnp.repeat(jnp.repeat(d2, 2, axis=1), 2, axis=2)
    d1 = conv3x3_s1(jnp.concatenate([u1, e1], axis=3),
                    pu['d1w'], pu['d1b'])
    mask = conv3x3_s1(d1, pu['ow'], pu['ob'],
                      act="sigmoid")[..., 0]

    refined = attention_refine(mask, pa['w1'], pa['b1'], pa['w2'], pa['b2'])
    r1p = conv3x3_c1_planar(refined, pb['w1'], pb['b1'])
    r1 = jnp.transpose(r1p, (0, 2, 3, 1))
    fmask = conv3x3_s1(r1, pb['w2'], pb['b2'], act="relu",
                       w2=pb['w3'], b2=pb['b3'],
                       act2="sigmoid")

    final_bbox = tail(f_gap, fmask[..., 0], ph, pe['fw'], pe['fb'])
    final_mask = jnp.transpose(fmask, (0, 3, 1, 2))
    return final_bbox, final_mask


# speedup vs baseline: 2.2359x; 2.2359x over previous
import jax
import jax.numpy as jnp
from jax import lax
from jax.experimental import pallas as pl
from jax.experimental.pallas import tpu as pltpu


def _row_tile(h):
    r = min(16, h)
    while h % r:
        r //= 2
    return r


def _act(x, act):
    if act == "relu":
        return jax.nn.relu(x)
    if act == "sigmoid":
        return jax.nn.sigmoid(x)
    return x


def _dimsem(n):
    return pltpu.CompilerParams(
        dimension_semantics=("parallel",) * n,
        vmem_limit_bytes=100 * 1024 * 1024,
    )


def conv3x3_s1(x, w, b, act="relu", w2=None, b2=None, act2=None):
    B_, H_, W_, C_ = x.shape
    O_ = w.shape[0]
    R = _row_tile(H_)
    T = H_ // R
    xp = jnp.pad(x, ((0, 0), (1, 1), (1, 1), (0, 0)))
    win = jnp.stack([xp[:, i * R:i * R + R + 2] for i in range(T)], axis=1)
    wk = jnp.transpose(w, (2, 3, 1, 0)).astype(x.dtype)
    bb = b.reshape(1, O_)
    fused = w2 is not None
    if fused:
        O2 = w2.shape[0]
        wk2 = w2.reshape(O2, O_).T.astype(x.dtype)
        bb2 = b2.reshape(1, O2)
    Oout = O2 if fused else O_

    def kern(xw, wr, br, *rest):
        if fused:
            w2r, b2r, out = rest
        else:
            (out,) = rest
        acc = jnp.zeros((R * W_, O_), jnp.float32)
        for dy in range(3):
            for dx in range(3):
                a = xw[0, 0, dy:dy + R, dx:dx + W_, :].reshape(R * W_, C_)
                acc = acc + jnp.dot(a, wr[dy, dx],
                                    preferred_element_type=jnp.float32)
        acc = _act(acc + br[...], act)
        if fused:
            acc = _act(jnp.dot(acc.astype(w2r.dtype), w2r[...],
                               preferred_element_type=jnp.float32) + b2r[...],
                       act2)
        out[0] = acc.astype(out.dtype).reshape(R, W_, Oout)

    in_specs = [
        pl.BlockSpec((1, 1, R + 2, W_ + 2, C_),
                     lambda bi, i: (bi, i, 0, 0, 0)),
        pl.BlockSpec((3, 3, C_, O_), lambda bi, i: (0, 0, 0, 0)),
        pl.BlockSpec((1, O_), lambda bi, i: (0, 0)),
    ]
    args = [win, wk, bb]
    if fused:
        in_specs.append(pl.BlockSpec((O_, O2), lambda bi, i: (0, 0)))
        in_specs.append(pl.BlockSpec((1, O2), lambda bi, i: (0, 0)))
        args += [wk2, bb2]
    return pl.pallas_call(
        kern,
        grid=(B_, T),
        in_specs=in_specs,
        out_specs=pl.BlockSpec((1, R, W_, Oout), lambda bi, i: (bi, i, 0, 0)),
        out_shape=jax.ShapeDtypeStruct((B_, H_, W_, Oout), jnp.float32),
        compiler_params=_dimsem(2),
    )(*args)


def conv3x3_s2(x, w, b):
    B_, H_, W_, C_ = x.shape
    O_ = w.shape[0]
    Ho, Wo = H_ // 2, W_ // 2
    Wp = Wo + 1
    R = _row_tile(Ho)
    xp = jnp.pad(x, ((0, 0), (0, 2), (0, 2), (0, 0)))
    xr = xp.reshape(B_, Ho + 1, 2, Wp, 2 * C_)
    ra = xr[:, 0:Ho]
    rb = xr[:, 1:Ho + 1]
    wk = jnp.transpose(w, (2, 3, 1, 0)).astype(x.dtype)
    bb = b.reshape(1, O_)

    def kern(a, b2, wr, br, out):
        rows = (a[0, :, 0], a[0, :, 1], b2[0, :, 0])
        acc = jnp.zeros((R * Wo, O_), jnp.float32)
        for dy in range(3):
            r = rows[dy]
            taps = (r[:, 0:Wo, 0:C_], r[:, 0:Wo, C_:2 * C_],
                    r[:, 1:Wo + 1, 0:C_])
            for dx in range(3):
                acc = acc + jnp.dot(taps[dx].reshape(R * Wo, C_), wr[dy, dx],
                                    preferred_element_type=jnp.float32)
        out[0] = jax.nn.relu(acc + br[...]).astype(out.dtype).reshape(R, Wo, O_)

    return pl.pallas_call(
        kern,
        grid=(B_, Ho // R),
        in_specs=[
            pl.BlockSpec((1, R, 2, Wp, 2 * C_), lambda bi, i: (bi, i, 0, 0, 0)),
            pl.BlockSpec((1, R, 2, Wp, 2 * C_), lambda bi, i: (bi, i, 0, 0, 0)),
            pl.BlockSpec((3, 3, C_, O_), lambda bi, i: (0, 0, 0, 0)),
            pl.BlockSpec((1, O_), lambda bi, i: (0, 0)),
        ],
        out_specs=pl.BlockSpec((1, R, Wo, O_), lambda bi, i: (bi, i, 0, 0)),
        out_shape=jax.ShapeDtypeStruct((B_, Ho, Wo, O_), jnp.float32),
        compiler_params=_dimsem(2),
    )(ra, rb, wk, bb)


def conv3x3_c1_planar(xg, w, b, out_planes=True):
    B_, H_, W_ = xg.shape
    O_ = w.shape[0]
    R = _row_tile(H_)
    xp = jnp.pad(xg, ((0, 0), (1, 1), (1, 1)))
    rows = [xp[:, d:d + H_] for d in range(3)]
    wf = w.reshape(O_, 9)

    def kern(x0, x1, x2, wr, br, out):
        srcs = (x0, x1, x2)
        taps = [srcs[dy][0][:, dx:dx + W_] for dy in range(3) for dx in range(3)]
        for o in range(O_):
            acc = taps[0] * wr[o, 0]
            for k in range(1, 9):
                acc = acc + taps[k] * wr[o, k]
            out[0, o] = jax.nn.relu(acc + br[o])

    return pl.pallas_call(
        kern,
        grid=(B_, H_ // R),
        in_specs=[pl.BlockSpec((1, R, W_ + 2), lambda bi, i: (bi, i, 0))
                  for _ in range(3)]
        + [pl.BlockSpec(memory_space=pltpu.SMEM),
           pl.BlockSpec(memory_space=pltpu.SMEM)],
        out_specs=pl.BlockSpec((1, O_, R, W_), lambda bi, i: (bi, 0, i, 0)),
        out_shape=jax.ShapeDtypeStruct((B_, O_, H_, W_), jnp.float32),
        compiler_params=_dimsem(2),
    )(rows[0], rows[1], rows[2], wf, b)


def attention_refine(mask, w1, b1, w2, b2):
    B_, H_, W_ = mask.shape
    O_ = w1.shape[0]
    R = _row_tile(H_)
    xp = jnp.pad(mask, ((0, 0), (1, 1), (1, 1)))
    rows = [xp[:, d:d + H_] for d in range(3)]
    wf = w1.reshape(O_, 9)
    w2f = w2.reshape(O_)

    def kern(x0, x1, x2, wr, br, w2r, b2r, out):
        srcs = (x0, x1, x2)
        taps = [srcs[dy][0][:, dx:dx + W_] for dy in range(3) for dx in range(3)]
        att = jnp.zeros((R, W_), jnp.float32) + b2r[0]
        for o in range(O_):
            acc = taps[0] * wr[o, 0]
            for k in range(1, 9):
                acc = acc + taps[k] * wr[o, k]
            att = att + jax.nn.relu(acc + br[o]) * w2r[o]
        out[0] = taps[4] * jax.nn.sigmoid(att)

    return pl.pallas_call(
        kern,
        grid=(B_, H_ // R),
        in_specs=[pl.BlockSpec((1, R, W_ + 2), lambda bi, i: (bi, i, 0))
                  for _ in range(3)]
        + [pl.BlockSpec(memory_space=pltpu.SMEM)] * 4,
        out_specs=pl.BlockSpec((1, R, W_), lambda bi, i: (bi, i, 0)),
        out_shape=jax.ShapeDtypeStruct((B_, H_, W_), jnp.float32),
        compiler_params=_dimsem(2),
    )(rows[0], rows[1], rows[2], wf, b1, w2f, b2)


def maxpool2(x):
    B_, H_, W_, C_ = x.shape
    Ho, Wo = H_ // 2, W_ // 2
    R = _row_tile(Ho)
    xr = x.reshape(B_, Ho, 2, Wo, 2 * C_)

    def kern(r, out):
        m = jnp.maximum(r[0, :, 0], r[0, :, 1])
        out[0] = jnp.maximum(m[:, :, 0:C_], m[:, :, C_:2 * C_])

    return pl.pallas_call(
        kern,
        grid=(B_, Ho // R),
        in_specs=[pl.BlockSpec((1, R, 2, Wo, 2 * C_),
                               lambda bi, i: (bi, i, 0, 0, 0))],
        out_specs=pl.BlockSpec((1, R, Wo, C_), lambda bi, i: (bi, i, 0, 0)),
        out_shape=jax.ShapeDtypeStruct((B_, Ho, Wo, C_), x.dtype),
        compiler_params=_dimsem(2),
    )(xr)


def tail(f_gap_in, mask, ph, pe_fw, pe_fb):
    B_, S_, F_ = f_gap_in.shape
    H_, W_ = mask.shape[1], mask.shape[2]
    fwT = pe_fw.T
    fbb = pe_fb.reshape(1, 4)
    w1T = ph['w1'].T
    w2T = ph['w2'].T
    w3T = ph['w3'].T
    b1 = ph['b1'].reshape(1, -1); g1 = ph['g1'].reshape(1, -1)
    be1 = ph['be1'].reshape(1, -1)
    b2 = ph['b2'].reshape(1, -1); g2 = ph['g2'].reshape(1, -1)
    be2 = ph['be2'].reshape(1, -1)
    b3 = ph['b3'].reshape(1, 4)

    def kern(fr, mr, fwr, fbr, w1r, b1r, g1r, be1r, w2r, b2r, g2r, be2r,
             w3r, b3r, out):
        gap = jnp.sum(fr[...], axis=1) * (1.0 / S_)
        bbox_feat = jax.nn.sigmoid(
            jnp.dot(gap, fwr[...], preferred_element_type=jnp.float32)
            + fbr[...])

        m = jnp.where(mr[...] > 0.5, 1.0, 0.0)
        rows = jnp.max(m, axis=2)
        cols = jnp.max(m, axis=1)
        ih = lax.broadcasted_iota(jnp.int32, (B_, H_), 1).astype(jnp.float32)
        iw = lax.broadcasted_iota(jnp.int32, (B_, W_), 1).astype(jnp.float32)
        big = jnp.float32(1e9)
        rmin = jnp.min(jnp.where(rows > 0.5, ih, big), axis=1, keepdims=True)
        rmax = jnp.max(jnp.where(rows > 0.5, ih, -big), axis=1, keepdims=True)
        cmin = jnp.min(jnp.where(cols > 0.5, iw, big), axis=1, keepdims=True)
        cmax = jnp.max(jnp.where(cols > 0.5, iw, -big), axis=1, keepdims=True)
        valid = (jnp.max(rows, axis=1, keepdims=True)
                 * jnp.max(cols, axis=1, keepdims=True)) > 0.5
        bbox = jnp.concatenate(
            [cmin * (1.0 / W_), rmin * (1.0 / H_),
             cmax * (1.0 / W_), rmax * (1.0 / H_)], axis=1)
        bbox = jnp.clip(bbox, 0.0, 1.0)
        mask_bbox = jnp.where(valid, bbox, 0.0)

        def bnorm(h, g, be):
            mu = jnp.mean(h, axis=0, keepdims=True)
            var = jnp.mean((h - mu) * (h - mu), axis=0, keepdims=True)
            return g * (h - mu) * jax.lax.rsqrt(var + 1e-5) + be

        hdd = jnp.concatenate([bbox_feat, mask_bbox], axis=1)
        h1 = bnorm(jax.nn.relu(
            jnp.dot(hdd, w1r[...], preferred_element_type=jnp.float32)
            + b1r[...]), g1r[...], be1r[...])
        h2 = bnorm(jax.nn.relu(
            jnp.dot(h1, w2r[...], preferred_element_type=jnp.float32)
            + b2r[...]), g2r[...], be2r[...])
        fb = jax.nn.sigmoid(
            jnp.dot(h2, w3r[...], preferred_element_type=jnp.float32)
            + b3r[...])
        x1 = fb[:, 0:1]; y1 = fb[:, 1:2]; x2 = fb[:, 2:3]; y2 = fb[:, 3:4]
        x2 = x1 + jax.nn.relu(x2 - x1) + 0.001
        y2 = y1 + jax.nn.relu(y2 - y1) + 0.001
        out[...] = jnp.concatenate([x1, y1, x2, y2], axis=1)

    return pl.pallas_call(
        kern,
        out_shape=jax.ShapeDtypeStruct((B_, 4), jnp.float32),
        compiler_params=pltpu.CompilerParams(
            vmem_limit_bytes=100 * 1024 * 1024),
    )(f_gap_in, mask, fwT, fbb, w1T, b1, g1, be1, w2T, b2, g2, be2, w3T, b3)


def kernel(x, params):
    B_, C_, H_, W_ = x.shape
    pe, pu = params['eff'], params['unet']
    pa, pb, ph = params['att'], params['br'], params['head']

    xt = jnp.transpose(x, (0, 2, 3, 1))
    f = conv3x3_s2(xt, pe['w1'], pe['b1'])
    f = conv3x3_s2(f, pe['w2'], pe['b2'])
    f = conv3x3_s2(f, pe['w3'], pe['b3'])
    S = f.shape[1] * f.shape[2]
    f_gap = f.reshape(B_, S, 128)

    xg = x[:, 0]
    e1p = conv3x3_c1_planar(xg, pu['e1w'], pu['e1b'])
    e1 = jnp.transpose(e1p, (0, 2, 3, 1))
    e2 = conv3x3_s1(maxpool2(e1), pu['e2w'], pu['e2b'])
    bt = conv3x3_s1(maxpool2(e2), pu['bw'], pu['bb'])
    u2 = jnp.repeat(jnp.repeat(bt, 2, axis=1), 2, axis=2)
    d2 = conv3x3_s1(jnp.concatenate([u2, e2], axis=3),
                    pu['d2w'], pu['d2b'])
    u1 = jnp.repeat(jnp.repeat(d2, 2, axis=1), 2, axis=2)
    d1 = conv3x3_s1(jnp.concatenate([u1, e1], axis=3),
                    pu['d1w'], pu['d1b'])
    mask = conv3x3_s1(d1, pu['ow'], pu['ob'],
                      act="sigmoid")[..., 0]

    refined = attention_refine(mask, pa['w1'], pa['b1'], pa['w2'], pa['b2'])
    r1p = conv3x3_c1_planar(refined, pb['w1'], pb['b1'])
    r1 = jnp.transpose(r1p, (0, 2, 3, 1))
    fmask = conv3x3_s1(r1, pb['w2'], pb['b2'], act="relu",
                       w2=pb['w3'], b2=pb['b3'],
                       act2="sigmoid")

    final_bbox = tail(f_gap, fmask[..., 0], ph, pe['fw'], pe['fb'])
    final_mask = jnp.transpose(fmask, (0, 3, 1, 2))
    return final_bbox, final_mask
